# Initial kernel scaffold; baseline (speedup 1.0000x reference)
import jax
import jax.numpy as jnp
from jax.experimental import pallas as pl


def kernel(outputs, labels):
    raise NotImplementedError("write your pallas kernel here")


# speedup vs baseline: 1.7027x; 1.7027x over previous
import functools

import jax
import jax.numpy as jnp
from jax.experimental import pallas as pl
from jax.experimental.pallas import tpu as pltpu

B = 16384
C = 1000
BR = 256
NB = B // BR


def _body(x_ref, lab_ref, out_ref, cnt_ref, ws_ref):
    i = pl.program_id(0)
    x = x_ref[...]
    m = jnp.max(x, axis=1, keepdims=True)
    e = jnp.exp(x - m)
    s = jnp.sum(e, axis=1, keepdims=True)
    q = 1.0 - e / s

    qmax = jnp.max(q, axis=1, keepdims=True)
    col = jax.lax.broadcasted_iota(jnp.int32, (BR, C), 1)
    idx = jnp.min(jnp.where(q == qmax, col, jnp.int32(C)), axis=1,
                  keepdims=True)

    lab = lab_ref[0]
    q_l = jnp.sum(jnp.where(col == lab, q, 0.0), axis=1, keepdims=True)
    a = -jnp.log(q_l) * qmax

    onehot = col == idx
    cnt_blk = jnp.sum(onehot.astype(jnp.float32), axis=0, keepdims=True)
    ws_blk = jnp.sum(jnp.where(onehot, a, 0.0), axis=0, keepdims=True)

    @pl.when(i == 0)
    def _():
        cnt_ref[...] = cnt_blk
        ws_ref[...] = ws_blk

    @pl.when(i > 0)
    def _():
        cnt_ref[...] += cnt_blk
        ws_ref[...] += ws_blk

    @pl.when(i == NB - 1)
    def _():
        cw = 1.0 - cnt_ref[...] * (1.0 / B)
        out_ref[...] = jnp.sum(ws_ref[...] * cw, axis=1, keepdims=True) * (1.0 / B)


@functools.partial(jax.jit, static_argnames=("interpret",))
def _run(outputs, labels, interpret=False):
    lab3 = labels.astype(jnp.int32).reshape(NB, BR, 1)
    loss = pl.pallas_call(
        _body,
        grid=(NB,),
        in_specs=[
            pl.BlockSpec((BR, C), lambda i: (i, 0)),
            pl.BlockSpec((1, BR, 1), lambda i: (i, 0, 0)),
        ],
        out_specs=pl.BlockSpec((1, 1), lambda i: (0, 0)),
        out_shape=jax.ShapeDtypeStruct((1, 1), jnp.float32),
        scratch_shapes=[
            pltpu.VMEM((1, C), jnp.float32),
            pltpu.VMEM((1, C), jnp.float32),
        ],
        interpret=interpret,
    )(outputs, lab3)
    return loss.reshape(())


def kernel(outputs, labels):
    return _run(outputs, labels)
